# baseline (speedup 1.0000x reference)
import jax
import jax.numpy as jnp
from jax.experimental import pallas as pl
from jax.experimental.pallas import tpu as pltpu

B, DIN = 16384, 768
H1, H2, D = 512, 256, 64
L, K = 3, 1024

BB = 1024
CH = 256


def _fused_kernel(x_ref, w1_ref, b1_ref, w2_ref, b2_ref, w3_ref, b3_ref,
                  cb_ref, cbhi_ref, cblo_ref, ids_ref, quant_ref, cn_ref):
    @pl.when(pl.program_id(0) == 0)
    def _():
        for l in range(L):
            cb = cb_ref[l]
            cn_ref[l] = jnp.sum(cb * cb, axis=1, keepdims=True).T

    x = x_ref[...]
    h = jnp.dot(x, w1_ref[...], preferred_element_type=jnp.float32)
    h = jnp.maximum(h + b1_ref[...], 0.0)
    h = jnp.dot(h, w2_ref[...], preferred_element_type=jnp.float32)
    h = jnp.maximum(h + b2_ref[...], 0.0)
    z = jnp.dot(h, w3_ref[...], preferred_element_type=jnp.float32)
    z = z + b3_ref[...]

    lane = jax.lax.broadcasted_iota(jnp.int32, (CH, K), 1)
    for c in range(BB // CH):
        res = z[c * CH:(c + 1) * CH]
        quant = jnp.zeros_like(res)
        ids = []
        for l in range(L):
            rn = jnp.sum(res * res, axis=1, keepdims=True)
            dotm = jax.lax.dot_general(res * (-2.0), cb_ref[l],
                                       (((1,), (1,)), ((), ())),
                                       preferred_element_type=jnp.float32)
            d2 = (rn + dotm) + cn_ref[l]
            mn = jnp.min(d2, axis=1, keepdims=True)
            idx = jnp.min(jnp.where(d2 == mn, lane, K), axis=1, keepdims=True)
            onehot = (lane == idx).astype(jnp.bfloat16)
            sel = (jnp.dot(onehot, cbhi_ref[l], preferred_element_type=jnp.float32)
                   + jnp.dot(onehot, cblo_ref[l], preferred_element_type=jnp.float32))
            quant = quant + sel
            res = res - sel
            ids.append(idx)
        ids_ref[c * CH:(c + 1) * CH, :] = jnp.concatenate(ids, axis=1)
        quant_ref[c * CH:(c + 1) * CH, :] = quant


@jax.jit
def kernel(x, W1, b1, W2, b2, W3, b3, codebooks):
    grid = (B // BB,)
    full = lambda *shape: pl.BlockSpec(shape, lambda i: (0,) * len(shape))
    cb_hi = codebooks.astype(jnp.bfloat16)
    cb_lo = (codebooks - cb_hi.astype(jnp.float32)).astype(jnp.bfloat16)
    sem_ids, quant = pl.pallas_call(
        _fused_kernel,
        grid=grid,
        in_specs=[
            pl.BlockSpec((BB, DIN), lambda i: (i, 0)),
            full(DIN, H1),
            full(1, H1),
            full(H1, H2),
            full(1, H2),
            full(H2, D),
            full(1, D),
            full(L, K, D),
            full(L, K, D),
            full(L, K, D),
        ],
        out_specs=[
            pl.BlockSpec((BB, L), lambda i: (i, 0)),
            pl.BlockSpec((BB, D), lambda i: (i, 0)),
        ],
        out_shape=[
            jax.ShapeDtypeStruct((B, L), jnp.int32),
            jax.ShapeDtypeStruct((B, D), jnp.float32),
        ],
        scratch_shapes=[pltpu.VMEM((L, 1, K), jnp.float32)],
    )(x, W1, b1.reshape(1, H1), W2, b2.reshape(1, H2), W3, b3.reshape(1, D),
      codebooks, cb_hi, cb_lo)
    return sem_ids, quant


# speedup vs baseline: 1.0099x; 1.0099x over previous
import jax
import jax.numpy as jnp
from jax.experimental import pallas as pl
from jax.experimental.pallas import tpu as pltpu

B, DIN = 16384, 768
H1, H2, D = 512, 256, 64
L, K = 3, 1024

BB = 2048
CH = 256


def _fused_kernel(x_ref, w1_ref, b1_ref, w2_ref, b2_ref, w3_ref, b3_ref,
                  cb_ref, cbhi_ref, cblo_ref, ids_ref, quant_ref, cn_ref):
    @pl.when(pl.program_id(0) == 0)
    def _():
        for l in range(L):
            cb = cb_ref[l]
            cn_ref[l] = jnp.sum(cb * cb, axis=1, keepdims=True).T

    x = x_ref[...]
    h = jnp.dot(x, w1_ref[...], preferred_element_type=jnp.float32)
    h = jnp.maximum(h + b1_ref[...], 0.0)
    h = jnp.dot(h, w2_ref[...], preferred_element_type=jnp.float32)
    h = jnp.maximum(h + b2_ref[...], 0.0)
    z = jnp.dot(h, w3_ref[...], preferred_element_type=jnp.float32)
    z = z + b3_ref[...]

    lane = jax.lax.broadcasted_iota(jnp.int32, (CH, K), 1)
    for c in range(BB // CH):
        res = z[c * CH:(c + 1) * CH]
        quant = jnp.zeros_like(res)
        ids = []
        for l in range(L):
            rn = jnp.sum(res * res, axis=1, keepdims=True)
            dotm = jax.lax.dot_general(res * (-2.0), cb_ref[l],
                                       (((1,), (1,)), ((), ())),
                                       preferred_element_type=jnp.float32)
            d2 = (rn + dotm) + cn_ref[l]
            mn = jnp.min(d2, axis=1, keepdims=True)
            idx = jnp.min(jnp.where(d2 == mn, lane, K), axis=1, keepdims=True)
            onehot = (lane == idx).astype(jnp.bfloat16)
            sel = (jnp.dot(onehot, cbhi_ref[l], preferred_element_type=jnp.float32)
                   + jnp.dot(onehot, cblo_ref[l], preferred_element_type=jnp.float32))
            quant = quant + sel
            res = res - sel
            ids.append(idx)
        ids_ref[c * CH:(c + 1) * CH, :] = jnp.concatenate(ids, axis=1)
        quant_ref[c * CH:(c + 1) * CH, :] = quant


@jax.jit
def kernel(x, W1, b1, W2, b2, W3, b3, codebooks):
    grid = (B // BB,)
    full = lambda *shape: pl.BlockSpec(shape, lambda i: (0,) * len(shape))
    cb_hi = codebooks.astype(jnp.bfloat16)
    cb_lo = (codebooks - cb_hi.astype(jnp.float32)).astype(jnp.bfloat16)
    sem_ids, quant = pl.pallas_call(
        _fused_kernel,
        grid=grid,
        in_specs=[
            pl.BlockSpec((BB, DIN), lambda i: (i, 0)),
            full(DIN, H1),
            full(1, H1),
            full(H1, H2),
            full(1, H2),
            full(H2, D),
            full(1, D),
            full(L, K, D),
            full(L, K, D),
            full(L, K, D),
        ],
        out_specs=[
            pl.BlockSpec((BB, L), lambda i: (i, 0)),
            pl.BlockSpec((BB, D), lambda i: (i, 0)),
        ],
        out_shape=[
            jax.ShapeDtypeStruct((B, L), jnp.int32),
            jax.ShapeDtypeStruct((B, D), jnp.float32),
        ],
        scratch_shapes=[pltpu.VMEM((L, 1, K), jnp.float32)],
    )(x, W1, b1.reshape(1, H1), W2, b2.reshape(1, H2), W3, b3.reshape(1, D),
      codebooks, cb_hi, cb_lo)
    return sem_ids, quant


# speedup vs baseline: 1.0137x; 1.0038x over previous
import jax
import jax.numpy as jnp
from jax.experimental import pallas as pl
from jax.experimental.pallas import tpu as pltpu

B, DIN = 16384, 768
H1, H2, D = 512, 256, 64
L, K = 3, 1024

BB = 2048
CH = 256


def _fused_kernel(x_ref, w1_ref, b1_ref, w2_ref, b2_ref, w3_ref, b3_ref,
                  cb_ref, ids_ref, quant_ref, cn_ref, cbhi_ref, cblo_ref):
    @pl.when(pl.program_id(0) == 0)
    def _():
        for l in range(L):
            cb = cb_ref[l]
            cn_ref[l] = jnp.sum(cb * cb, axis=1, keepdims=True).T
            hi = cb.astype(jnp.bfloat16)
            cbhi_ref[l] = hi
            cblo_ref[l] = (cb - hi.astype(jnp.float32)).astype(jnp.bfloat16)

    x = x_ref[...]
    h = jnp.dot(x, w1_ref[...], preferred_element_type=jnp.float32)
    h = jnp.maximum(h + b1_ref[...], 0.0)
    h = jnp.dot(h, w2_ref[...], preferred_element_type=jnp.float32)
    h = jnp.maximum(h + b2_ref[...], 0.0)
    z = jnp.dot(h, w3_ref[...], preferred_element_type=jnp.float32)
    z = z + b3_ref[...]

    lane = jax.lax.broadcasted_iota(jnp.int32, (CH, K), 1)
    for c in range(BB // CH):
        res = z[c * CH:(c + 1) * CH]
        quant = jnp.zeros_like(res)
        ids = []
        for l in range(L):
            rn = jnp.sum(res * res, axis=1, keepdims=True)
            dotm = jax.lax.dot_general(res * (-2.0), cb_ref[l],
                                       (((1,), (1,)), ((), ())),
                                       preferred_element_type=jnp.float32)
            d2 = (rn + dotm) + cn_ref[l]
            mn = jnp.min(d2, axis=1, keepdims=True)
            idx = jnp.min(jnp.where(d2 == mn, lane, K), axis=1, keepdims=True)
            onehot = (lane == idx).astype(jnp.bfloat16)
            sel = (jnp.dot(onehot, cbhi_ref[l], preferred_element_type=jnp.float32)
                   + jnp.dot(onehot, cblo_ref[l], preferred_element_type=jnp.float32))
            quant = quant + sel
            res = res - sel
            ids.append(idx)
        ids_ref[c * CH:(c + 1) * CH, :] = jnp.concatenate(ids, axis=1)
        quant_ref[c * CH:(c + 1) * CH, :] = quant


@jax.jit
def kernel(x, W1, b1, W2, b2, W3, b3, codebooks):
    grid = (B // BB,)
    full = lambda *shape: pl.BlockSpec(shape, lambda i: (0,) * len(shape))
    sem_ids, quant = pl.pallas_call(
        _fused_kernel,
        grid=grid,
        in_specs=[
            pl.BlockSpec((BB, DIN), lambda i: (i, 0)),
            full(DIN, H1),
            full(1, H1),
            full(H1, H2),
            full(1, H2),
            full(H2, D),
            full(1, D),
            full(L, K, D),
        ],
        out_specs=[
            pl.BlockSpec((BB, L), lambda i: (i, 0)),
            pl.BlockSpec((BB, D), lambda i: (i, 0)),
        ],
        out_shape=[
            jax.ShapeDtypeStruct((B, L), jnp.int32),
            jax.ShapeDtypeStruct((B, D), jnp.float32),
        ],
        scratch_shapes=[
            pltpu.VMEM((L, 1, K), jnp.float32),
            pltpu.VMEM((L, K, D), jnp.bfloat16),
            pltpu.VMEM((L, K, D), jnp.bfloat16),
        ],
    )(x, W1, b1.reshape(1, H1), W2, b2.reshape(1, H2), W3, b3.reshape(1, D),
      codebooks)
    return sem_ids, quant


# speedup vs baseline: 1.3390x; 1.3209x over previous
import jax
import jax.numpy as jnp
from jax.experimental import pallas as pl
from jax.experimental.pallas import tpu as pltpu

B, DIN = 16384, 768
H1, H2, D = 512, 256, 64
L, K = 3, 1024

BB = 2048
CH = 256
NC = BB // CH


def _fused_kernel(x_ref, w1_ref, b1_ref, w2_ref, b2_ref, w3_ref, b3_ref,
                  cb_ref, ids_ref, quant_ref, cn_ref, cbhi_ref, cblo_ref):
    @pl.when(pl.program_id(0) == 0)
    def _():
        for l in range(L):
            cb = cb_ref[l]
            cn_ref[l] = jnp.sum(cb * cb, axis=1, keepdims=True).T
            hi = cb.astype(jnp.bfloat16)
            cbhi_ref[l] = hi
            cblo_ref[l] = (cb - hi.astype(jnp.float32)).astype(jnp.bfloat16)

    x = x_ref[...]
    h = jnp.dot(x, w1_ref[...], preferred_element_type=jnp.float32)
    h = jnp.maximum(h + b1_ref[...], 0.0)
    h = jnp.dot(h, w2_ref[...], preferred_element_type=jnp.float32)
    h = jnp.maximum(h + b2_ref[...], 0.0)
    z = jnp.dot(h, w3_ref[...], preferred_element_type=jnp.float32)
    z = z + b3_ref[...]

    lane = jax.lax.broadcasted_iota(jnp.int32, (CH, K), 1)
    res = [z[c * CH:(c + 1) * CH] for c in range(NC)]
    quant = [jnp.zeros((CH, D), jnp.float32) for _ in range(NC)]
    ids = [[None] * L for _ in range(NC)]
    d2s = [None] * NC
    oh = [None] * NC

    n_stages = 3 * L
    for t in range(n_stages + NC - 1):
        for c in range(NC):
            s = t - c
            if s < 0 or s >= n_stages:
                continue
            l, phase = divmod(s, 3)
            if phase == 0:
                r = res[c]
                rn = jnp.sum(r * r, axis=1, keepdims=True)
                dotm = jax.lax.dot_general(r * (-2.0), cb_ref[l],
                                           (((1,), (1,)), ((), ())),
                                           preferred_element_type=jnp.float32)
                d2s[c] = (rn + dotm) + cn_ref[l]
            elif phase == 1:
                d2 = d2s[c]
                mn = jnp.min(d2, axis=1, keepdims=True)
                idx = jnp.min(jnp.where(d2 == mn, lane, K), axis=1,
                              keepdims=True)
                ids[c][l] = idx
                oh[c] = (lane == idx).astype(jnp.bfloat16)
            else:
                onehot = oh[c]
                sel = (jnp.dot(onehot, cbhi_ref[l],
                               preferred_element_type=jnp.float32)
                       + jnp.dot(onehot, cblo_ref[l],
                                 preferred_element_type=jnp.float32))
                quant[c] = quant[c] + sel
                res[c] = res[c] - sel

    for c in range(NC):
        ids_ref[c * CH:(c + 1) * CH, :] = jnp.concatenate(ids[c], axis=1)
        quant_ref[c * CH:(c + 1) * CH, :] = quant[c]


@jax.jit
def kernel(x, W1, b1, W2, b2, W3, b3, codebooks):
    grid = (B // BB,)
    full = lambda *shape: pl.BlockSpec(shape, lambda i: (0,) * len(shape))
    sem_ids, quant = pl.pallas_call(
        _fused_kernel,
        grid=grid,
        in_specs=[
            pl.BlockSpec((BB, DIN), lambda i: (i, 0)),
            full(DIN, H1),
            full(1, H1),
            full(H1, H2),
            full(1, H2),
            full(H2, D),
            full(1, D),
            full(L, K, D),
        ],
        out_specs=[
            pl.BlockSpec((BB, L), lambda i: (i, 0)),
            pl.BlockSpec((BB, D), lambda i: (i, 0)),
        ],
        out_shape=[
            jax.ShapeDtypeStruct((B, L), jnp.int32),
            jax.ShapeDtypeStruct((B, D), jnp.float32),
        ],
        scratch_shapes=[
            pltpu.VMEM((L, 1, K), jnp.float32),
            pltpu.VMEM((L, K, D), jnp.bfloat16),
            pltpu.VMEM((L, K, D), jnp.bfloat16),
        ],
    )(x, W1, b1.reshape(1, H1), W2, b2.reshape(1, H2), W3, b3.reshape(1, D),
      codebooks)
    return sem_ids, quant


# speedup vs baseline: 1.4065x; 1.0504x over previous
import jax
import jax.numpy as jnp
from jax.experimental import pallas as pl
from jax.experimental.pallas import tpu as pltpu

B, DIN = 16384, 768
H1, H2, D = 512, 256, 64
L, K = 3, 1024

BB = 2048
CH = 256
NC = BB // CH
NSTAGES = 3 + 3 * L


def _fused_kernel(x_ref, w1_ref, b1_ref, w2_ref, b2_ref, w3_ref, b3_ref,
                  cb_ref, ids_ref, quant_ref, cn_ref, cbhi_ref, cblo_ref):
    @pl.when(pl.program_id(0) == 0)
    def _():
        for l in range(L):
            cb = cb_ref[l]
            cn_ref[l] = jnp.sum(cb * cb, axis=1, keepdims=True).T
            hi = cb.astype(jnp.bfloat16)
            cbhi_ref[l] = hi
            cblo_ref[l] = (cb - hi.astype(jnp.float32)).astype(jnp.bfloat16)

    lane = jax.lax.broadcasted_iota(jnp.int32, (CH, K), 1)
    h = [None] * NC
    res = [None] * NC
    quant = [jnp.zeros((CH, D), jnp.float32) for _ in range(NC)]
    ids = [[None] * L for _ in range(NC)]
    d2s = [None] * NC
    oh = [None] * NC

    for t in range(NSTAGES + NC - 1):
        for c in range(NC):
            s = t - c
            if s < 0 or s >= NSTAGES:
                continue
            if s == 0:
                xc = x_ref[c * CH:(c + 1) * CH, :]
                hc = jnp.dot(xc, w1_ref[...],
                             preferred_element_type=jnp.float32)
                h[c] = jnp.maximum(hc + b1_ref[...], 0.0)
            elif s == 1:
                hc = jnp.dot(h[c], w2_ref[...],
                             preferred_element_type=jnp.float32)
                h[c] = jnp.maximum(hc + b2_ref[...], 0.0)
            elif s == 2:
                zc = jnp.dot(h[c], w3_ref[...],
                             preferred_element_type=jnp.float32)
                res[c] = zc + b3_ref[...]
            else:
                l, phase = divmod(s - 3, 3)
                if phase == 0:
                    r = res[c]
                    rn = jnp.sum(r * r, axis=1, keepdims=True)
                    dotm = jax.lax.dot_general(r * (-2.0), cb_ref[l],
                                               (((1,), (1,)), ((), ())),
                                               preferred_element_type=jnp.float32)
                    d2s[c] = (rn + dotm) + cn_ref[l]
                elif phase == 1:
                    d2 = d2s[c]
                    mn = jnp.min(d2, axis=1, keepdims=True)
                    idx = jnp.min(jnp.where(d2 == mn, lane, K), axis=1,
                                  keepdims=True)
                    ids[c][l] = idx
                    oh[c] = (lane == idx).astype(jnp.bfloat16)
                else:
                    onehot = oh[c]
                    sel = (jnp.dot(onehot, cbhi_ref[l],
                                   preferred_element_type=jnp.float32)
                           + jnp.dot(onehot, cblo_ref[l],
                                     preferred_element_type=jnp.float32))
                    quant[c] = quant[c] + sel
                    res[c] = res[c] - sel

    for c in range(NC):
        ids_ref[c * CH:(c + 1) * CH, :] = jnp.concatenate(ids[c], axis=1)
        quant_ref[c * CH:(c + 1) * CH, :] = quant[c]


@jax.jit
def kernel(x, W1, b1, W2, b2, W3, b3, codebooks):
    grid = (B // BB,)
    full = lambda *shape: pl.BlockSpec(shape, lambda i: (0,) * len(shape))
    sem_ids, quant = pl.pallas_call(
        _fused_kernel,
        grid=grid,
        in_specs=[
            pl.BlockSpec((BB, DIN), lambda i: (i, 0)),
            full(DIN, H1),
            full(1, H1),
            full(H1, H2),
            full(1, H2),
            full(H2, D),
            full(1, D),
            full(L, K, D),
        ],
        out_specs=[
            pl.BlockSpec((BB, L), lambda i: (i, 0)),
            pl.BlockSpec((BB, D), lambda i: (i, 0)),
        ],
        out_shape=[
            jax.ShapeDtypeStruct((B, L), jnp.int32),
            jax.ShapeDtypeStruct((B, D), jnp.float32),
        ],
        scratch_shapes=[
            pltpu.VMEM((L, 1, K), jnp.float32),
            pltpu.VMEM((L, K, D), jnp.bfloat16),
            pltpu.VMEM((L, K, D), jnp.bfloat16),
        ],
    )(x, W1, b1.reshape(1, H1), W2, b2.reshape(1, H2), W3, b3.reshape(1, D),
      codebooks)
    return sem_ids, quant
